# Initial kernel scaffold; baseline (speedup 1.0000x reference)
import jax
import jax.numpy as jnp
from jax.experimental import pallas as pl


def kernel(x, edge_index, edge_weight, in_conv_w, in_conv_b, b0_in_bn_g, b0_in_bn_b, b0_in_conv_w, b0_in_conv_b, b0_h_bn_g, b0_h_bn_b, b0_h_conv_w, b0_h_conv_b, b1_in_bn_g, b1_in_bn_b, b1_in_conv_w, b1_in_conv_b, b1_h_bn_g, b1_h_bn_b, b1_h_conv_w, b1_h_conv_b, out_bn_g, out_bn_b, lin_w, lin_b):
    raise NotImplementedError("write your pallas kernel here")


# speedup vs baseline: 1.5760x; 1.5760x over previous
import functools

import jax
import jax.numpy as jnp
from jax import lax
from jax.experimental import pallas as pl
from jax.experimental.pallas import tpu as pltpu
from jax.experimental.pallas import tpu_sc as plsc

V = 10000
E = 160000
B = 4
C = 32
F = B * C
NW = 32
RPW = 320
VPAD = NW * RPW
CH_E = 128
EPAD = ((E + CH_E - 1) // CH_E + 1) * CH_E
RBLK = 256
NBLK = VPAD // RBLK
FBLK = 64


def _spmm_body(recur, xf_hbm, prev_hbm, rs_hbm, cs_hbm, ws_hbm, offp_hbm,
               y_hbm, offs_v, idx_v, rs_v, ws_v, rows_v, acc_v, pbuf_v, sem):
    cid = lax.axis_index("c")
    sid = lax.axis_index("s")
    wid = sid * 2 + cid
    base_row = wid * RPW

    pltpu.sync_copy(offp_hbm.at[wid], offs_v)
    ov = offs_v[pl.ds(0, 16)]
    off0 = ov[0]
    off1 = ov[1]
    astart = lax.bitwise_and(off0, -8)
    nch = lax.shift_right_arithmetic(off1 - astart + (CH_E - 1), 7)
    nch = jnp.maximum(nch, 0)

    def zbody(i, c):
        acc_v[pl.ds(i * 16, 16)] = jnp.zeros((16,), jnp.float32)
        return c
    lax.fori_loop(0, RPW * F // 16, zbody, 0)

    def chunk_body(ch, c):
        st = pl.multiple_of(astart + ch * CH_E, 8)
        pltpu.sync_copy(cs_hbm.at[pl.ds(st, CH_E)], idx_v)
        pltpu.sync_copy(rs_hbm.at[pl.ds(st, CH_E)], rs_v.at[pl.ds(0, CH_E)])
        pltpu.sync_copy(ws_hbm.at[pl.ds(st, CH_E)], ws_v.at[pl.ds(0, CH_E)])
        pltpu.async_copy(xf_hbm.at[idx_v], rows_v, sem).wait()

        def edge_body(e, c2):
            ge = st + e
            ok = jnp.logical_and(ge >= off0, ge < off1)
            r = rs_v[pl.ds(e, 16)][0] - base_row
            rc = jnp.where(ok, r, 0)
            wv = jnp.where(ok, ws_v[pl.ds(e, 16)][0], 0.0)
            ob = rc * F
            for j in range(F // 16):
                xr = rows_v[e, pl.ds(j * 16, 16)]
                plsc.addupdate(acc_v.at[pl.ds(ob + j * 16, 16)], xr * wv)
            return c2
        lax.fori_loop(0, CH_E, edge_body, 0)
        return c
    lax.fori_loop(0, nch, chunk_body, 0)

    for blk in range(RPW // FBLK):
        rb = base_row + blk * FBLK
        if recur:
            pltpu.sync_copy(prev_hbm.at[pl.ds(rb, FBLK)], pbuf_v)

        def fbody(i, c):
            for j in range(F // 16):
                a = acc_v[pl.ds((blk * FBLK + i) * F + j * 16, 16)]
                if recur:
                    pbuf_v[i, pl.ds(j * 16, 16)] = 2.0 * a - pbuf_v[i, pl.ds(j * 16, 16)]
                else:
                    pbuf_v[i, pl.ds(j * 16, 16)] = a
            return c
        lax.fori_loop(0, FBLK, fbody, 0)
        pltpu.sync_copy(pbuf_v, y_hbm.at[pl.ds(rb, FBLK)])


def _make_spmm(recur):
    mesh = plsc.VectorSubcoreMesh(core_axis_name="c", subcore_axis_name="s")
    return pl.kernel(
        functools.partial(_spmm_body, recur),
        out_type=jax.ShapeDtypeStruct((VPAD, F), jnp.float32),
        mesh=mesh,
        scratch_types=[
            pltpu.VMEM((16,), jnp.int32),
            pltpu.VMEM((CH_E,), jnp.int32),
            pltpu.VMEM((CH_E + 16,), jnp.int32),
            pltpu.VMEM((CH_E + 16,), jnp.float32),
            pltpu.VMEM((CH_E, F), jnp.float32),
            pltpu.VMEM((RPW * F,), jnp.float32),
            pltpu.VMEM((FBLK, F), jnp.float32),
            pltpu.SemaphoreType.DMA,
        ],
        name=f"sc_spmm_recur{int(recur)}",
    )


_spmm_first = _make_spmm(False)
_spmm_recur = _make_spmm(True)


def _combine_body(has_res, refs):
    if has_res:
        (t0, t1, t2, t3, wbd, bias, res, out, s1, s2, mx) = refs
    else:
        (t0, t1, t2, t3, wbd, bias, out, s1, s2, mx) = refs
    o = bias[...]
    o = o + jnp.dot(t0[...], wbd[0], preferred_element_type=jnp.float32)
    o = o + jnp.dot(t1[...], wbd[1], preferred_element_type=jnp.float32)
    o = o + jnp.dot(t2[...], wbd[2], preferred_element_type=jnp.float32)
    o = o + jnp.dot(t3[...], wbd[3], preferred_element_type=jnp.float32)
    if has_res:
        o = o + res[...]
    o = jnp.maximum(o, 0.0)
    out[...] = o

    i = pl.program_id(0)
    rowid = lax.broadcasted_iota(jnp.int32, (RBLK, 1), 0) + i * RBLK
    valid = rowid < V
    om = jnp.where(valid, o, 0.0)
    ps1 = jnp.sum(om, axis=0, keepdims=True)
    ps2 = jnp.sum(om * om, axis=0, keepdims=True)
    pmx = jnp.max(jnp.where(valid, o, -jnp.inf), axis=0, keepdims=True)

    @pl.when(i == 0)
    def _():
        s1[...] = jnp.zeros_like(s1)
        s2[...] = jnp.zeros_like(s2)
        mx[...] = jnp.full_like(mx, -jnp.inf)

    s1[...] += ps1
    s2[...] += ps2
    mx[...] = jnp.maximum(mx[...], pmx)


def _make_combine(has_res):
    blk = pl.BlockSpec((RBLK, F), lambda i: (i, 0))
    small = pl.BlockSpec((1, F), lambda i: (0, 0))
    in_specs = [blk, blk, blk, blk,
                pl.BlockSpec((4, F, F), lambda i: (0, 0, 0)), small]
    if has_res:
        in_specs.append(blk)
    return pl.pallas_call(
        lambda *refs: _combine_body(has_res, refs),
        grid=(NBLK,),
        in_specs=in_specs,
        out_specs=[blk, small, small, small],
        out_shape=[jax.ShapeDtypeStruct((VPAD, F), jnp.float32),
                   jax.ShapeDtypeStruct((1, F), jnp.float32),
                   jax.ShapeDtypeStruct((1, F), jnp.float32),
                   jax.ShapeDtypeStruct((1, F), jnp.float32)],
        name=f"tc_combine_res{int(has_res)}",
    )


_combine_plain = _make_combine(False)
_combine_res = _make_combine(True)


def _bnapply_body(x, scale, shift, out):
    out[...] = x[...] * scale[...] + shift[...]


_bnapply = pl.pallas_call(
    _bnapply_body,
    grid=(NBLK,),
    in_specs=[pl.BlockSpec((RBLK, F), lambda i: (i, 0)),
              pl.BlockSpec((1, F), lambda i: (0, 0)),
              pl.BlockSpec((1, F), lambda i: (0, 0))],
    out_specs=pl.BlockSpec((RBLK, F), lambda i: (i, 0)),
    out_shape=jax.ShapeDtypeStruct((VPAD, F), jnp.float32),
    name="tc_bnapply",
)


def _head_body(mx, g, b, lw, lb, out):
    xs = [mx[:, pl.ds(i * C, C)] for i in range(B)]
    m = (xs[0] + xs[1] + xs[2] + xs[3]) * 0.25
    v = (xs[0] * xs[0] + xs[1] * xs[1] + xs[2] * xs[2] + xs[3] * xs[3]) * 0.25
    v = v - m * m
    inv = lax.rsqrt(v + 1e-5)
    rows = []
    for i in range(B):
        xh = (xs[i] - m) * inv * g[...] + b[...]
        logit = jnp.dot(xh, lw[...], preferred_element_type=jnp.float32) + lb[...]
        logit = jnp.maximum(logit, 0.0)
        z = logit - jnp.max(logit, axis=1, keepdims=True)
        rows.append(z - jnp.log(jnp.sum(jnp.exp(z), axis=1, keepdims=True)))
    out[...] = jnp.concatenate(rows, axis=0)


_head = pl.pallas_call(
    _head_body,
    out_shape=jax.ShapeDtypeStruct((B, 10), jnp.float32),
    name="tc_head",
)


def _cheb(xf, rs, cs, ws, offp, wbd, bias, res):
    t1 = _spmm_first(xf, xf, rs, cs, ws, offp)
    t2 = _spmm_recur(t1, xf, rs, cs, ws, offp)
    t3 = _spmm_recur(t2, t1, rs, cs, ws, offp)
    if res is None:
        return _combine_plain(xf, t1, t2, t3, wbd, bias)
    return _combine_res(xf, t1, t2, t3, wbd, bias, res)


def _bn_scale_shift(s1, s2, g, b):
    cnt = float(B * V)
    s1f = s1.reshape(B, C).sum(axis=0)
    s2f = s2.reshape(B, C).sum(axis=0)
    m = s1f / cnt
    var = s2f / cnt - m * m
    inv = lax.rsqrt(var + 1e-5)
    scale = inv * g
    shift = b - m * scale
    return (jnp.tile(scale, B).reshape(1, F),
            jnp.tile(shift, B).reshape(1, F))


def kernel(x, edge_index, edge_weight, in_conv_w, in_conv_b,
           b0_in_bn_g, b0_in_bn_b, b0_in_conv_w, b0_in_conv_b,
           b0_h_bn_g, b0_h_bn_b, b0_h_conv_w, b0_h_conv_b,
           b1_in_bn_g, b1_in_bn_b, b1_in_conv_w, b1_in_conv_b,
           b1_h_bn_g, b1_h_bn_b, b1_h_conv_w, b1_h_conv_b,
           out_bn_g, out_bn_b, lin_w, lin_b):
    row = edge_index[0].astype(jnp.int32)
    col = edge_index[1].astype(jnp.int32)
    ew = edge_weight

    deg = jax.ops.segment_sum(ew, row, num_segments=V)
    dis = lax.rsqrt(jnp.maximum(deg, 1e-12))
    wn = -dis[row] * ew * dis[col]

    perm = jnp.argsort(row)
    rs = jnp.concatenate([row[perm], jnp.full((EPAD - E,), VPAD, jnp.int32)])
    cs = jnp.concatenate([col[perm], jnp.zeros((EPAD - E,), jnp.int32)])
    ws = jnp.concatenate([wn[perm], jnp.zeros((EPAD - E,), jnp.float32)])
    bounds = jnp.arange(0, VPAD + 1, RPW, dtype=jnp.int32)
    off = jnp.searchsorted(rs[:E], bounds).astype(jnp.int32)
    offp = jnp.zeros((NW, 16), jnp.int32)
    offp = offp.at[:, 0].set(off[:-1]).at[:, 1].set(off[1:])

    xf = jnp.transpose(x, (2, 0, 1)).reshape(V, F)
    xf = jnp.concatenate([xf, jnp.zeros((VPAD - V, F), jnp.float32)])

    eye = jnp.eye(B, dtype=jnp.float32)
    def bd(w):
        return jnp.einsum('ab,kcd->kacbd', eye, w).reshape(w.shape[0], F, F)

    def bias128(bv):
        return jnp.tile(bv, B).reshape(1, F)

    out0, s1, s2, _ = _cheb(xf, rs, cs, ws, offp, bd(in_conv_w),
                            bias128(in_conv_b), None)
    sc, sh = _bn_scale_shift(s1, s2, b0_in_bn_g, b0_in_bn_b)
    xn = _bnapply(out0, sc, sh)
    c1, s1, s2, _ = _cheb(xn, rs, cs, ws, offp, bd(b0_in_conv_w),
                          bias128(b0_in_conv_b), None)
    sc, sh = _bn_scale_shift(s1, s2, b0_h_bn_g, b0_h_bn_b)
    c1n = _bnapply(c1, sc, sh)
    out1, s1, s2, _ = _cheb(c1n, rs, cs, ws, offp, bd(b0_h_conv_w),
                            bias128(b0_h_conv_b), xn)
    sc, sh = _bn_scale_shift(s1, s2, b1_in_bn_g, b1_in_bn_b)
    xn = _bnapply(out1, sc, sh)
    c1, s1, s2, _ = _cheb(xn, rs, cs, ws, offp, bd(b1_in_conv_w),
                          bias128(b1_in_conv_b), None)
    sc, sh = _bn_scale_shift(s1, s2, b1_h_bn_g, b1_h_bn_b)
    c1n = _bnapply(c1, sc, sh)
    _, _, _, mx = _cheb(c1n, rs, cs, ws, offp, bd(b1_h_conv_w),
                        bias128(b1_h_conv_b), xn)
    return _head(mx, out_bn_g.reshape(1, C), out_bn_b.reshape(1, C),
                 lin_w, lin_b.reshape(1, 10))


# speedup vs baseline: 1.9585x; 1.2427x over previous
import functools

import jax
import jax.numpy as jnp
from jax import lax
from jax.experimental import pallas as pl
from jax.experimental.pallas import tpu as pltpu
from jax.experimental.pallas import tpu_sc as plsc

V = 10000
E = 160000
B = 4
C = 32
F = B * C
NW = 32
RPW = 320
VPAD = NW * RPW
CH_E = 256
EPAD = E + 2 * CH_E
RBLK = 256
NBLK = VPAD // RBLK
FBLK = 64


def _spmm_body(recur, xf_hbm, prev_hbm, cs_hbm, eo_hbm, ew_hbm, offp_hbm,
               y_hbm, offs_v, ia_v, ib_v, oa_v, ob_v, wa_v, wb_v, ra_v, rb_v,
               acc_v, pbuf_v, g0, g1, si0, si1, se0, se1):
    cid = lax.axis_index("c")
    sid = lax.axis_index("s")
    wid = sid * 2 + cid
    base_row = wid * RPW

    pltpu.sync_copy(offp_hbm.at[wid], offs_v)
    ov = offs_v[pl.ds(0, 16)]
    off0 = ov[0]
    off1 = ov[1]
    astart = lax.bitwise_and(off0, -8)
    nch = lax.shift_right_arithmetic(off1 - astart + (CH_E - 1), 8)
    nch = jnp.maximum(nch, 0)

    ibufs = (ia_v, ib_v)
    obufs = (oa_v, ob_v)
    wbufs = (wa_v, wb_v)
    rbufs = (ra_v, rb_v)
    gsems = (g0, g1)
    isems = (si0, si1)
    esems = (se0, se1)

    def chstart(ch):
        return pl.multiple_of(astart + ch * CH_E, 8)

    def start_idx(ch, b, sem):
        pltpu.async_copy(cs_hbm.at[pl.ds(chstart(ch), CH_E)], ibufs[b], sem)

    def start_edat(ch, b, sem):
        st = chstart(ch)
        pltpu.async_copy(eo_hbm.at[pl.ds(st, CH_E)],
                         obufs[b].at[pl.ds(0, CH_E)], sem)
        pltpu.async_copy(ew_hbm.at[pl.ds(st, CH_E)],
                         wbufs[b].at[pl.ds(0, CH_E)], sem)

    def start_gather(b, sem):
        for h in range(CH_E // 128):
            pltpu.async_copy(
                xf_hbm.at[ibufs[b].at[pl.ds(h * 128, 128)]],
                rbufs[b].at[pl.ds(h * 128, 128)], sem)

    def wait_gather(b, sem):
        for h in range(CH_E // 128):
            pltpu.make_async_copy(
                xf_hbm.at[ibufs[b].at[pl.ds(h * 128, 128)]],
                rbufs[b].at[pl.ds(h * 128, 128)], sem).wait()

    def wait_small(ch, b, sem, ed):
        st = chstart(ch)
        if ed:
            pltpu.make_async_copy(eo_hbm.at[pl.ds(st, CH_E)],
                                  obufs[b].at[pl.ds(0, CH_E)], sem).wait()
            pltpu.make_async_copy(ew_hbm.at[pl.ds(st, CH_E)],
                                  wbufs[b].at[pl.ds(0, CH_E)], sem).wait()
        else:
            pltpu.make_async_copy(cs_hbm.at[pl.ds(st, CH_E)],
                                  ibufs[b], sem).wait()

    def zbody(i, c):
        for u in range(8):
            acc_v[pl.ds(i * 128 + u * 16, 16)] = jnp.zeros((16,), jnp.float32)
        return c
    lax.fori_loop(0, RPW * F // 128, zbody, 0)

    @pl.when(nch > 0)
    def _():
        pltpu.sync_copy(cs_hbm.at[pl.ds(chstart(0), CH_E)], ia_v)
        start_gather(0, g0)
        start_edat(0, 0, se0)

        @pl.when(nch > 1)
        def _():
            start_idx(1, 1, si1)

    def pair_body(g, c):
        for b in range(2):
            ch = g * 2 + b
            st = chstart(ch)

            @pl.when(ch < nch)
            def _():
                @pl.when(ch + 1 < nch)
                def _():
                    wait_small(ch + 1, 1 - b, isems[1 - b], ed=False)
                    start_gather(1 - b, gsems[1 - b])
                wait_gather(b, gsems[b])

                @pl.when(ch + 2 < nch)
                def _():
                    start_idx(ch + 2, b, isems[b])

                @pl.when(ch + 1 < nch)
                def _():
                    start_edat(ch + 1, 1 - b, esems[1 - b])
                wait_small(ch, b, esems[b], ed=True)

                o_cur = obufs[b]
                w_cur = wbufs[b]
                r_cur = rbufs[b]

                def egroup(gi, c2):
                    for u in range(4):
                        e = gi * 4 + u
                        ge = st + e
                        ok = jnp.logical_and(ge >= off0, ge < off1)
                        ob = jnp.where(ok, o_cur[pl.ds(e, 16)][0], 0)
                        wv = jnp.where(ok, w_cur[pl.ds(e, 16)][0], 0.0)
                        for j in range(F // 16):
                            xr = r_cur[e, pl.ds(j * 16, 16)]
                            plsc.addupdate(acc_v.at[pl.ds(ob + j * 16, 16)],
                                           xr * wv)
                    return c2
                lax.fori_loop(0, CH_E // 4, egroup, 0)
        return c
    lax.fori_loop(0, lax.shift_right_arithmetic(nch + 1, 1), pair_body, 0)

    for blk in range(RPW // FBLK):
        rb = base_row + blk * FBLK
        if recur:
            pltpu.sync_copy(prev_hbm.at[pl.ds(rb, FBLK)], pbuf_v)

        def fbody(i, c):
            for j in range(F // 16):
                a = acc_v[pl.ds((blk * FBLK + i) * F + j * 16, 16)]
                if recur:
                    pbuf_v[i, pl.ds(j * 16, 16)] = 2.0 * a - pbuf_v[i, pl.ds(j * 16, 16)]
                else:
                    pbuf_v[i, pl.ds(j * 16, 16)] = a
            return c
        lax.fori_loop(0, FBLK, fbody, 0)
        pltpu.sync_copy(pbuf_v, y_hbm.at[pl.ds(rb, FBLK)])


def _make_spmm(recur):
    mesh = plsc.VectorSubcoreMesh(core_axis_name="c", subcore_axis_name="s")
    return pl.kernel(
        functools.partial(_spmm_body, recur),
        out_type=jax.ShapeDtypeStruct((VPAD, F), jnp.float32),
        mesh=mesh,
        scratch_types=[
            pltpu.VMEM((16,), jnp.int32),
            pltpu.VMEM((CH_E,), jnp.int32),
            pltpu.VMEM((CH_E,), jnp.int32),
            pltpu.VMEM((CH_E + 16,), jnp.int32),
            pltpu.VMEM((CH_E + 16,), jnp.int32),
            pltpu.VMEM((CH_E + 16,), jnp.float32),
            pltpu.VMEM((CH_E + 16,), jnp.float32),
            pltpu.VMEM((CH_E, F), jnp.float32),
            pltpu.VMEM((CH_E, F), jnp.float32),
            pltpu.VMEM((RPW * F,), jnp.float32),
            pltpu.VMEM((FBLK, F), jnp.float32),
            pltpu.SemaphoreType.DMA,
            pltpu.SemaphoreType.DMA,
            pltpu.SemaphoreType.DMA,
            pltpu.SemaphoreType.DMA,
            pltpu.SemaphoreType.DMA,
            pltpu.SemaphoreType.DMA,
        ],
        name=f"sc_spmm_recur{int(recur)}",
    )


_spmm_first = _make_spmm(False)
_spmm_recur = _make_spmm(True)


def _combine_body(has_res, refs):
    if has_res:
        (t0, t1, t2, t3, wbd, bias, res, out, s1, s2, mx) = refs
    else:
        (t0, t1, t2, t3, wbd, bias, out, s1, s2, mx) = refs
    o = bias[...]
    o = o + jnp.dot(t0[...], wbd[0], preferred_element_type=jnp.float32)
    o = o + jnp.dot(t1[...], wbd[1], preferred_element_type=jnp.float32)
    o = o + jnp.dot(t2[...], wbd[2], preferred_element_type=jnp.float32)
    o = o + jnp.dot(t3[...], wbd[3], preferred_element_type=jnp.float32)
    if has_res:
        o = o + res[...]
    o = jnp.maximum(o, 0.0)
    out[...] = o

    i = pl.program_id(0)
    rowid = lax.broadcasted_iota(jnp.int32, (RBLK, 1), 0) + i * RBLK
    valid = rowid < V
    om = jnp.where(valid, o, 0.0)
    ps1 = jnp.sum(om, axis=0, keepdims=True)
    ps2 = jnp.sum(om * om, axis=0, keepdims=True)
    pmx = jnp.max(jnp.where(valid, o, -jnp.inf), axis=0, keepdims=True)

    @pl.when(i == 0)
    def _():
        s1[...] = jnp.zeros_like(s1)
        s2[...] = jnp.zeros_like(s2)
        mx[...] = jnp.full_like(mx, -jnp.inf)

    s1[...] += ps1
    s2[...] += ps2
    mx[...] = jnp.maximum(mx[...], pmx)


def _make_combine(has_res):
    blk = pl.BlockSpec((RBLK, F), lambda i: (i, 0))
    small = pl.BlockSpec((1, F), lambda i: (0, 0))
    in_specs = [blk, blk, blk, blk,
                pl.BlockSpec((4, F, F), lambda i: (0, 0, 0)), small]
    if has_res:
        in_specs.append(blk)
    return pl.pallas_call(
        lambda *refs: _combine_body(has_res, refs),
        grid=(NBLK,),
        in_specs=in_specs,
        out_specs=[blk, small, small, small],
        out_shape=[jax.ShapeDtypeStruct((VPAD, F), jnp.float32),
                   jax.ShapeDtypeStruct((1, F), jnp.float32),
                   jax.ShapeDtypeStruct((1, F), jnp.float32),
                   jax.ShapeDtypeStruct((1, F), jnp.float32)],
        name=f"tc_combine_res{int(has_res)}",
    )


_combine_plain = _make_combine(False)
_combine_res = _make_combine(True)


def _bnapply_body(x, scale, shift, out):
    out[...] = x[...] * scale[...] + shift[...]


_bnapply = pl.pallas_call(
    _bnapply_body,
    grid=(NBLK,),
    in_specs=[pl.BlockSpec((RBLK, F), lambda i: (i, 0)),
              pl.BlockSpec((1, F), lambda i: (0, 0)),
              pl.BlockSpec((1, F), lambda i: (0, 0))],
    out_specs=pl.BlockSpec((RBLK, F), lambda i: (i, 0)),
    out_shape=jax.ShapeDtypeStruct((VPAD, F), jnp.float32),
    name="tc_bnapply",
)


def _head_body(mx, g, b, lw, lb, out):
    xs = [mx[:, pl.ds(i * C, C)] for i in range(B)]
    m = (xs[0] + xs[1] + xs[2] + xs[3]) * 0.25
    v = (xs[0] * xs[0] + xs[1] * xs[1] + xs[2] * xs[2] + xs[3] * xs[3]) * 0.25
    v = v - m * m
    inv = lax.rsqrt(v + 1e-5)
    rows = []
    for i in range(B):
        xh = (xs[i] - m) * inv * g[...] + b[...]
        logit = jnp.dot(xh, lw[...], preferred_element_type=jnp.float32) + lb[...]
        logit = jnp.maximum(logit, 0.0)
        z = logit - jnp.max(logit, axis=1, keepdims=True)
        rows.append(z - jnp.log(jnp.sum(jnp.exp(z), axis=1, keepdims=True)))
    out[...] = jnp.concatenate(rows, axis=0)


_head = pl.pallas_call(
    _head_body,
    out_shape=jax.ShapeDtypeStruct((B, 10), jnp.float32),
    name="tc_head",
)


def _cheb(xf, cs, eo, ew, offp, wbd, bias, res):
    t1 = _spmm_first(xf, xf, cs, eo, ew, offp)
    t2 = _spmm_recur(t1, xf, cs, eo, ew, offp)
    t3 = _spmm_recur(t2, t1, cs, eo, ew, offp)
    if res is None:
        return _combine_plain(xf, t1, t2, t3, wbd, bias)
    return _combine_res(xf, t1, t2, t3, wbd, bias, res)


def _bn_scale_shift(s1, s2, g, b):
    cnt = float(B * V)
    s1f = s1.reshape(B, C).sum(axis=0)
    s2f = s2.reshape(B, C).sum(axis=0)
    m = s1f / cnt
    var = s2f / cnt - m * m
    inv = lax.rsqrt(var + 1e-5)
    scale = inv * g
    shift = b - m * scale
    return (jnp.tile(scale, B).reshape(1, F),
            jnp.tile(shift, B).reshape(1, F))


def kernel(x, edge_index, edge_weight, in_conv_w, in_conv_b,
           b0_in_bn_g, b0_in_bn_b, b0_in_conv_w, b0_in_conv_b,
           b0_h_bn_g, b0_h_bn_b, b0_h_conv_w, b0_h_conv_b,
           b1_in_bn_g, b1_in_bn_b, b1_in_conv_w, b1_in_conv_b,
           b1_h_bn_g, b1_h_bn_b, b1_h_conv_w, b1_h_conv_b,
           out_bn_g, out_bn_b, lin_w, lin_b):
    row = edge_index[0].astype(jnp.int32)
    col = edge_index[1].astype(jnp.int32)
    ew = edge_weight

    deg = jax.ops.segment_sum(ew, row, num_segments=V)
    dis = lax.rsqrt(jnp.maximum(deg, 1e-12))
    wn = -dis[row] * ew * dis[col]

    perm = jnp.argsort(row)
    rs = row[perm]
    cs = jnp.concatenate([col[perm], jnp.zeros((EPAD - E,), jnp.int32)])
    eo = jnp.concatenate([(rs % RPW) * F, jnp.zeros((EPAD - E,), jnp.int32)])
    ew = jnp.concatenate([wn[perm], jnp.zeros((EPAD - E,), jnp.float32)])
    bounds = jnp.arange(0, VPAD + 1, RPW, dtype=jnp.int32)
    off = jnp.searchsorted(rs, bounds).astype(jnp.int32)
    offp = jnp.zeros((NW, 16), jnp.int32)
    offp = offp.at[:, 0].set(off[:-1]).at[:, 1].set(off[1:])

    xf = jnp.transpose(x, (2, 0, 1)).reshape(V, F)
    xf = jnp.concatenate([xf, jnp.zeros((VPAD - V, F), jnp.float32)])

    eye = jnp.eye(B, dtype=jnp.float32)
    def bd(w):
        return jnp.einsum('ab,kcd->kacbd', eye, w).reshape(w.shape[0], F, F)

    def bias128(bv):
        return jnp.tile(bv, B).reshape(1, F)

    out0, s1, s2, _ = _cheb(xf, cs, eo, ew, offp, bd(in_conv_w),
                            bias128(in_conv_b), None)
    sc, sh = _bn_scale_shift(s1, s2, b0_in_bn_g, b0_in_bn_b)
    xn = _bnapply(out0, sc, sh)
    c1, s1, s2, _ = _cheb(xn, cs, eo, ew, offp, bd(b0_in_conv_w),
                          bias128(b0_in_conv_b), None)
    sc, sh = _bn_scale_shift(s1, s2, b0_h_bn_g, b0_h_bn_b)
    c1n = _bnapply(c1, sc, sh)
    out1, s1, s2, _ = _cheb(c1n, cs, eo, ew, offp, bd(b0_h_conv_w),
                            bias128(b0_h_conv_b), xn)
    sc, sh = _bn_scale_shift(s1, s2, b1_in_bn_g, b1_in_bn_b)
    xn = _bnapply(out1, sc, sh)
    c1, s1, s2, _ = _cheb(xn, cs, eo, ew, offp, bd(b1_in_conv_w),
                          bias128(b1_in_conv_b), None)
    sc, sh = _bn_scale_shift(s1, s2, b1_h_bn_g, b1_h_bn_b)
    c1n = _bnapply(c1, sc, sh)
    _, _, _, mx = _cheb(c1n, cs, eo, ew, offp, bd(b1_h_conv_w),
                        bias128(b1_h_conv_b), xn)
    return _head(mx, out_bn_g.reshape(1, C), out_bn_b.reshape(1, C),
                 lin_w, lin_b.reshape(1, 10))


# speedup vs baseline: 3.2248x; 1.6466x over previous
import functools

import jax
import jax.numpy as jnp
from jax import lax
from jax.experimental import pallas as pl
from jax.experimental.pallas import tpu as pltpu
from jax.experimental.pallas import tpu_sc as plsc

V = 10000
E = 160000
B = 4
C = 32
F = B * C
NW = 32
RPW = 320
VPAD = NW * RPW
CH_E = 256
EPAD = E + 2 * CH_E
RBLK = 256
NBLK = VPAD // RBLK
FBLK = 64


def _spmm_body(recur, xf_hbm, prev_hbm, cs_hbm, eo_hbm, ew_hbm, offp_hbm,
               y_hbm, offs_v, ia_v, ib_v, oa_v, ob_v, wa_v, wb_v, ra_v, rb_v,
               acc_v, pbuf_v, g0, g1, si0, si1, se0, se1):
    cid = lax.axis_index("c")
    sid = lax.axis_index("s")
    wid = sid * 2 + cid
    base_row = wid * RPW

    pltpu.sync_copy(offp_hbm.at[wid], offs_v)
    ov = offs_v[pl.ds(0, 16)]
    off0 = ov[0]
    off1 = ov[1]
    astart = lax.bitwise_and(off0, -8)
    nch = lax.shift_right_arithmetic(off1 - astart + (CH_E - 1), 8)
    nch = jnp.maximum(nch, 0)

    ibufs = (ia_v, ib_v)
    obufs = (oa_v, ob_v)
    wbufs = (wa_v, wb_v)
    rbufs = (ra_v, rb_v)
    gsems = (g0, g1)
    isems = (si0, si1)
    esems = (se0, se1)

    def chstart(ch):
        return pl.multiple_of(astart + ch * CH_E, 8)

    def start_idx(ch, b, sem):
        pltpu.async_copy(cs_hbm.at[pl.ds(chstart(ch), CH_E)], ibufs[b], sem)

    def start_edat(ch, b, sem):
        st = chstart(ch)
        pltpu.async_copy(eo_hbm.at[pl.ds(st, CH_E)],
                         obufs[b].at[pl.ds(0, CH_E)], sem)
        pltpu.async_copy(ew_hbm.at[pl.ds(st, CH_E)],
                         wbufs[b].at[pl.ds(0, CH_E)], sem)

    def start_gather(b, sem):
        for h in range(CH_E // 128):
            pltpu.async_copy(
                xf_hbm.at[ibufs[b].at[pl.ds(h * 128, 128)]],
                rbufs[b].at[pl.ds(h * 128, 128)], sem)

    def wait_gather(b, sem):
        for h in range(CH_E // 128):
            pltpu.make_async_copy(
                xf_hbm.at[ibufs[b].at[pl.ds(h * 128, 128)]],
                rbufs[b].at[pl.ds(h * 128, 128)], sem).wait()

    def wait_small(ch, b, sem, ed):
        st = chstart(ch)
        if ed:
            pltpu.make_async_copy(eo_hbm.at[pl.ds(st, CH_E)],
                                  obufs[b].at[pl.ds(0, CH_E)], sem).wait()
            pltpu.make_async_copy(ew_hbm.at[pl.ds(st, CH_E)],
                                  wbufs[b].at[pl.ds(0, CH_E)], sem).wait()
        else:
            pltpu.make_async_copy(cs_hbm.at[pl.ds(st, CH_E)],
                                  ibufs[b], sem).wait()

    def zbody(i, c):
        for u in range(8):
            acc_v[pl.ds(i * 128 + u * 16, 16)] = jnp.zeros((16,), jnp.float32)
        return c
    lax.fori_loop(0, RPW * F // 128, zbody, 0)

    @pl.when(nch > 0)
    def _():
        pltpu.sync_copy(cs_hbm.at[pl.ds(chstart(0), CH_E)], ia_v)
        start_gather(0, g0)
        start_edat(0, 0, se0)

        @pl.when(nch > 1)
        def _():
            start_idx(1, 1, si1)

    def pair_body(g, c):
        for b in range(2):
            ch = g * 2 + b
            st = chstart(ch)

            @pl.when(ch < nch)
            def _():
                @pl.when(ch + 1 < nch)
                def _():
                    wait_small(ch + 1, 1 - b, isems[1 - b], ed=False)
                    start_gather(1 - b, gsems[1 - b])
                wait_gather(b, gsems[b])

                @pl.when(ch + 2 < nch)
                def _():
                    start_idx(ch + 2, b, isems[b])

                @pl.when(ch + 1 < nch)
                def _():
                    start_edat(ch + 1, 1 - b, esems[1 - b])
                wait_small(ch, b, esems[b], ed=True)

                o_cur = obufs[b]
                w_cur = wbufs[b]
                r_cur = rbufs[b]

                def egroup(gi, c2):
                    EG = 4
                    ovs = [o_cur[pl.ds(gi * EG + u, 16)] for u in range(EG)]
                    wvs = [w_cur[pl.ds(gi * EG + u, 16)] for u in range(EG)]
                    scs = []
                    for u in range(EG):
                        ge = st + gi * EG + u
                        ok = jnp.logical_and(ge >= off0, ge < off1)
                        scs.append((jnp.where(ok, ovs[u][0], 0),
                                    jnp.where(ok, wvs[u][0], 0.0)))
                    xs = [[r_cur[gi * EG + u, pl.ds(j * 16, 16)]
                           for j in range(F // 16)] for u in range(EG)]
                    for u in range(EG):
                        ob_u, wv_u = scs[u]
                        for j in range(F // 16):
                            plsc.addupdate(acc_v.at[pl.ds(ob_u + j * 16, 16)],
                                           xs[u][j] * wv_u)
                    return c2
                lax.fori_loop(0, CH_E // 4, egroup, 0)
        return c
    lax.fori_loop(0, lax.shift_right_arithmetic(nch + 1, 1), pair_body, 0)

    for blk in range(RPW // FBLK):
        rb = base_row + blk * FBLK
        if recur:
            pltpu.sync_copy(prev_hbm.at[pl.ds(rb, FBLK)], pbuf_v)

        def fbody(i, c):
            for j in range(F // 16):
                a = acc_v[pl.ds((blk * FBLK + i) * F + j * 16, 16)]
                if recur:
                    pbuf_v[i, pl.ds(j * 16, 16)] = 2.0 * a - pbuf_v[i, pl.ds(j * 16, 16)]
                else:
                    pbuf_v[i, pl.ds(j * 16, 16)] = a
            return c
        lax.fori_loop(0, FBLK, fbody, 0)
        pltpu.sync_copy(pbuf_v, y_hbm.at[pl.ds(rb, FBLK)])


def _make_spmm(recur):
    mesh = plsc.VectorSubcoreMesh(core_axis_name="c", subcore_axis_name="s")
    return pl.kernel(
        functools.partial(_spmm_body, recur),
        out_type=jax.ShapeDtypeStruct((VPAD, F), jnp.float32),
        mesh=mesh,
        scratch_types=[
            pltpu.VMEM((16,), jnp.int32),
            pltpu.VMEM((CH_E,), jnp.int32),
            pltpu.VMEM((CH_E,), jnp.int32),
            pltpu.VMEM((CH_E + 16,), jnp.int32),
            pltpu.VMEM((CH_E + 16,), jnp.int32),
            pltpu.VMEM((CH_E + 16,), jnp.float32),
            pltpu.VMEM((CH_E + 16,), jnp.float32),
            pltpu.VMEM((CH_E, F), jnp.float32),
            pltpu.VMEM((CH_E, F), jnp.float32),
            pltpu.VMEM((RPW * F,), jnp.float32),
            pltpu.VMEM((FBLK, F), jnp.float32),
            pltpu.SemaphoreType.DMA,
            pltpu.SemaphoreType.DMA,
            pltpu.SemaphoreType.DMA,
            pltpu.SemaphoreType.DMA,
            pltpu.SemaphoreType.DMA,
            pltpu.SemaphoreType.DMA,
        ],
        name=f"sc_spmm_recur{int(recur)}",
    )


_spmm_first = _make_spmm(False)
_spmm_recur = _make_spmm(True)


def _combine_body(has_res, refs):
    if has_res:
        (t0, t1, t2, t3, wbd, bias, res, out, s1, s2, mx) = refs
    else:
        (t0, t1, t2, t3, wbd, bias, out, s1, s2, mx) = refs
    o = bias[...]
    o = o + jnp.dot(t0[...], wbd[0], preferred_element_type=jnp.float32)
    o = o + jnp.dot(t1[...], wbd[1], preferred_element_type=jnp.float32)
    o = o + jnp.dot(t2[...], wbd[2], preferred_element_type=jnp.float32)
    o = o + jnp.dot(t3[...], wbd[3], preferred_element_type=jnp.float32)
    if has_res:
        o = o + res[...]
    o = jnp.maximum(o, 0.0)
    out[...] = o

    i = pl.program_id(0)
    rowid = lax.broadcasted_iota(jnp.int32, (RBLK, 1), 0) + i * RBLK
    valid = rowid < V
    om = jnp.where(valid, o, 0.0)
    ps1 = jnp.sum(om, axis=0, keepdims=True)
    ps2 = jnp.sum(om * om, axis=0, keepdims=True)
    pmx = jnp.max(jnp.where(valid, o, -jnp.inf), axis=0, keepdims=True)

    @pl.when(i == 0)
    def _():
        s1[...] = jnp.zeros_like(s1)
        s2[...] = jnp.zeros_like(s2)
        mx[...] = jnp.full_like(mx, -jnp.inf)

    s1[...] += ps1
    s2[...] += ps2
    mx[...] = jnp.maximum(mx[...], pmx)


def _make_combine(has_res):
    blk = pl.BlockSpec((RBLK, F), lambda i: (i, 0))
    small = pl.BlockSpec((1, F), lambda i: (0, 0))
    in_specs = [blk, blk, blk, blk,
                pl.BlockSpec((4, F, F), lambda i: (0, 0, 0)), small]
    if has_res:
        in_specs.append(blk)
    return pl.pallas_call(
        lambda *refs: _combine_body(has_res, refs),
        grid=(NBLK,),
        in_specs=in_specs,
        out_specs=[blk, small, small, small],
        out_shape=[jax.ShapeDtypeStruct((VPAD, F), jnp.float32),
                   jax.ShapeDtypeStruct((1, F), jnp.float32),
                   jax.ShapeDtypeStruct((1, F), jnp.float32),
                   jax.ShapeDtypeStruct((1, F), jnp.float32)],
        name=f"tc_combine_res{int(has_res)}",
    )


_combine_plain = _make_combine(False)
_combine_res = _make_combine(True)


def _bnapply_body(x, scale, shift, out):
    out[...] = x[...] * scale[...] + shift[...]


_bnapply = pl.pallas_call(
    _bnapply_body,
    grid=(NBLK,),
    in_specs=[pl.BlockSpec((RBLK, F), lambda i: (i, 0)),
              pl.BlockSpec((1, F), lambda i: (0, 0)),
              pl.BlockSpec((1, F), lambda i: (0, 0))],
    out_specs=pl.BlockSpec((RBLK, F), lambda i: (i, 0)),
    out_shape=jax.ShapeDtypeStruct((VPAD, F), jnp.float32),
    name="tc_bnapply",
)


def _head_body(mx, g, b, lw, lb, out):
    xs = [mx[:, pl.ds(i * C, C)] for i in range(B)]
    m = (xs[0] + xs[1] + xs[2] + xs[3]) * 0.25
    v = (xs[0] * xs[0] + xs[1] * xs[1] + xs[2] * xs[2] + xs[3] * xs[3]) * 0.25
    v = v - m * m
    inv = lax.rsqrt(v + 1e-5)
    rows = []
    for i in range(B):
        xh = (xs[i] - m) * inv * g[...] + b[...]
        logit = jnp.dot(xh, lw[...], preferred_element_type=jnp.float32) + lb[...]
        logit = jnp.maximum(logit, 0.0)
        z = logit - jnp.max(logit, axis=1, keepdims=True)
        rows.append(z - jnp.log(jnp.sum(jnp.exp(z), axis=1, keepdims=True)))
    out[...] = jnp.concatenate(rows, axis=0)


_head = pl.pallas_call(
    _head_body,
    out_shape=jax.ShapeDtypeStruct((B, 10), jnp.float32),
    name="tc_head",
)


def _cheb(xf, cs, eo, ew, offp, wbd, bias, res):
    t1 = _spmm_first(xf, xf, cs, eo, ew, offp)
    t2 = _spmm_recur(t1, xf, cs, eo, ew, offp)
    t3 = _spmm_recur(t2, t1, cs, eo, ew, offp)
    if res is None:
        return _combine_plain(xf, t1, t2, t3, wbd, bias)
    return _combine_res(xf, t1, t2, t3, wbd, bias, res)


def _bn_scale_shift(s1, s2, g, b):
    cnt = float(B * V)
    s1f = s1.reshape(B, C).sum(axis=0)
    s2f = s2.reshape(B, C).sum(axis=0)
    m = s1f / cnt
    var = s2f / cnt - m * m
    inv = lax.rsqrt(var + 1e-5)
    scale = inv * g
    shift = b - m * scale
    return (jnp.tile(scale, B).reshape(1, F),
            jnp.tile(shift, B).reshape(1, F))


def kernel(x, edge_index, edge_weight, in_conv_w, in_conv_b,
           b0_in_bn_g, b0_in_bn_b, b0_in_conv_w, b0_in_conv_b,
           b0_h_bn_g, b0_h_bn_b, b0_h_conv_w, b0_h_conv_b,
           b1_in_bn_g, b1_in_bn_b, b1_in_conv_w, b1_in_conv_b,
           b1_h_bn_g, b1_h_bn_b, b1_h_conv_w, b1_h_conv_b,
           out_bn_g, out_bn_b, lin_w, lin_b):
    row = edge_index[0].astype(jnp.int32)
    col = edge_index[1].astype(jnp.int32)
    ew = edge_weight

    deg = jax.ops.segment_sum(ew, row, num_segments=V)
    dis = lax.rsqrt(jnp.maximum(deg, 1e-12))
    wn = -dis[row] * ew * dis[col]

    perm = jnp.argsort(row)
    rs = row[perm]
    cs = jnp.concatenate([col[perm], jnp.zeros((EPAD - E,), jnp.int32)])
    eo = jnp.concatenate([(rs % RPW) * F, jnp.zeros((EPAD - E,), jnp.int32)])
    ew = jnp.concatenate([wn[perm], jnp.zeros((EPAD - E,), jnp.float32)])
    bounds = jnp.arange(0, VPAD + 1, RPW, dtype=jnp.int32)
    off = jnp.searchsorted(rs, bounds).astype(jnp.int32)
    offp = jnp.zeros((NW, 16), jnp.int32)
    offp = offp.at[:, 0].set(off[:-1]).at[:, 1].set(off[1:])

    xf = jnp.transpose(x, (2, 0, 1)).reshape(V, F)
    xf = jnp.concatenate([xf, jnp.zeros((VPAD - V, F), jnp.float32)])

    eye = jnp.eye(B, dtype=jnp.float32)
    def bd(w):
        return jnp.einsum('ab,kcd->kacbd', eye, w).reshape(w.shape[0], F, F)

    def bias128(bv):
        return jnp.tile(bv, B).reshape(1, F)

    out0, s1, s2, _ = _cheb(xf, cs, eo, ew, offp, bd(in_conv_w),
                            bias128(in_conv_b), None)
    sc, sh = _bn_scale_shift(s1, s2, b0_in_bn_g, b0_in_bn_b)
    xn = _bnapply(out0, sc, sh)
    c1, s1, s2, _ = _cheb(xn, cs, eo, ew, offp, bd(b0_in_conv_w),
                          bias128(b0_in_conv_b), None)
    sc, sh = _bn_scale_shift(s1, s2, b0_h_bn_g, b0_h_bn_b)
    c1n = _bnapply(c1, sc, sh)
    out1, s1, s2, _ = _cheb(c1n, cs, eo, ew, offp, bd(b0_h_conv_w),
                            bias128(b0_h_conv_b), xn)
    sc, sh = _bn_scale_shift(s1, s2, b1_in_bn_g, b1_in_bn_b)
    xn = _bnapply(out1, sc, sh)
    c1, s1, s2, _ = _cheb(xn, cs, eo, ew, offp, bd(b1_in_conv_w),
                          bias128(b1_in_conv_b), None)
    sc, sh = _bn_scale_shift(s1, s2, b1_h_bn_g, b1_h_bn_b)
    c1n = _bnapply(c1, sc, sh)
    _, _, _, mx = _cheb(c1n, cs, eo, ew, offp, bd(b1_h_conv_w),
                        bias128(b1_h_conv_b), xn)
    return _head(mx, out_bn_g.reshape(1, C), out_bn_b.reshape(1, C),
                 lin_w, lin_b.reshape(1, 10))


# speedup vs baseline: 7.4508x; 2.3105x over previous
import functools

import jax
import jax.numpy as jnp
from jax import lax
from jax.experimental import pallas as pl
from jax.experimental.pallas import tpu as pltpu
from jax.experimental.pallas import tpu_sc as plsc

V = 10000
E = 160000
B = 4
C = 32
F = B * C
NW = 32
RPW = 320
VPAD = NW * RPW
CH_E = 256
EPAD = E + 2 * CH_E
RBLK = 256
NBLK = VPAD // RBLK
FBLK = 64


def _spmm_body(recur, xf_hbm, prev_hbm, cs_hbm, eo_hbm, ew_hbm, offp_hbm,
               y_hbm, offs_v, ia_v, ib_v, oa_v, ob_v, wa_v, wb_v, ra_v, rb_v,
               acc_v, pbuf_v, g0, g1, si0, si1, se0, se1):
    cid = lax.axis_index("c")
    sid = lax.axis_index("s")
    wid = sid * 2 + cid
    base_row = wid * RPW

    pltpu.sync_copy(offp_hbm.at[wid], offs_v)
    ov = offs_v[pl.ds(0, 16)]
    off0 = ov[0]
    off1 = ov[1]
    astart = lax.bitwise_and(off0, -8)
    nch = lax.shift_right_arithmetic(off1 - astart + (CH_E - 1), 8)
    nch = jnp.maximum(nch, 0)

    ibufs = (ia_v, ib_v)
    obufs = (oa_v, ob_v)
    wbufs = (wa_v, wb_v)
    rbufs = (ra_v, rb_v)
    gsems = (g0, g1)
    isems = (si0, si1)
    esems = (se0, se1)

    def chstart(ch):
        return pl.multiple_of(astart + ch * CH_E, 8)

    def start_idx(ch, b, sem):
        pltpu.async_copy(cs_hbm.at[pl.ds(chstart(ch), CH_E)], ibufs[b], sem)

    def start_edat(ch, b, sem):
        st = chstart(ch)
        pltpu.async_copy(eo_hbm.at[pl.ds(st, CH_E)],
                         obufs[b].at[pl.ds(0, CH_E)], sem)
        pltpu.async_copy(ew_hbm.at[pl.ds(st, CH_E)],
                         wbufs[b].at[pl.ds(0, CH_E)], sem)

    def start_gather(b, sem):
        for h in range(CH_E // 128):
            pltpu.async_copy(
                xf_hbm.at[ibufs[b].at[pl.ds(h * 128, 128)]],
                rbufs[b].at[pl.ds(h * 128, 128)], sem)

    def wait_gather(b, sem):
        for h in range(CH_E // 128):
            pltpu.make_async_copy(
                xf_hbm.at[ibufs[b].at[pl.ds(h * 128, 128)]],
                rbufs[b].at[pl.ds(h * 128, 128)], sem).wait()

    def wait_small(ch, b, sem, ed):
        st = chstart(ch)
        if ed:
            pltpu.make_async_copy(eo_hbm.at[pl.ds(st, CH_E)],
                                  obufs[b].at[pl.ds(0, CH_E)], sem).wait()
            pltpu.make_async_copy(ew_hbm.at[pl.ds(st, CH_E)],
                                  wbufs[b].at[pl.ds(0, CH_E)], sem).wait()
        else:
            pltpu.make_async_copy(cs_hbm.at[pl.ds(st, CH_E)],
                                  ibufs[b], sem).wait()

    def zbody(i, c):
        for u in range(8):
            acc_v[pl.ds(i * 128 + u * 16, 16)] = jnp.zeros((16,), jnp.float32)
        return c
    lax.fori_loop(0, RPW * F // 128, zbody, 0)

    @pl.when(nch > 0)
    def _():
        pltpu.sync_copy(cs_hbm.at[pl.ds(chstart(0), CH_E)], ia_v)
        start_gather(0, g0)
        start_edat(0, 0, se0)

        @pl.when(nch > 1)
        def _():
            start_idx(1, 1, si1)

    def pair_body(g, c):
        for b in range(2):
            ch = g * 2 + b
            st = chstart(ch)

            @pl.when(ch < nch)
            def _():
                @pl.when(ch + 1 < nch)
                def _():
                    wait_small(ch + 1, 1 - b, isems[1 - b], ed=False)
                    start_gather(1 - b, gsems[1 - b])
                wait_gather(b, gsems[b])

                @pl.when(ch + 2 < nch)
                def _():
                    start_idx(ch + 2, b, isems[b])

                @pl.when(ch + 1 < nch)
                def _():
                    start_edat(ch + 1, 1 - b, esems[1 - b])
                wait_small(ch, b, esems[b], ed=True)

                o_cur = obufs[b]
                w_cur = wbufs[b]
                r_cur = rbufs[b]

                def egroup(gi, c2):
                    EG = 4
                    ovs = [o_cur[pl.ds(gi * EG + u, 16)] for u in range(EG)]
                    wvs = [w_cur[pl.ds(gi * EG + u, 16)] for u in range(EG)]
                    scs = []
                    for u in range(EG):
                        ge = st + gi * EG + u
                        ok = jnp.logical_and(ge >= off0, ge < off1)
                        scs.append((jnp.where(ok, ovs[u][0], 0),
                                    jnp.where(ok, wvs[u][0], 0.0)))
                    xs = [[r_cur[gi * EG + u, pl.ds(j * 16, 16)]
                           for j in range(F // 16)] for u in range(EG)]
                    for u in range(EG):
                        ob_u, wv_u = scs[u]
                        for j in range(F // 16):
                            plsc.addupdate(acc_v.at[pl.ds(ob_u + j * 16, 16)],
                                           xs[u][j] * wv_u)
                    return c2
                lax.fori_loop(0, CH_E // 4, egroup, 0)
        return c
    lax.fori_loop(0, lax.shift_right_arithmetic(nch + 1, 1), pair_body, 0)

    for blk in range(RPW // FBLK):
        rb = base_row + blk * FBLK
        if recur:
            pltpu.sync_copy(prev_hbm.at[pl.ds(rb, FBLK)], pbuf_v)

        def fbody(i, c):
            for j in range(F // 16):
                a = acc_v[pl.ds((blk * FBLK + i) * F + j * 16, 16)]
                if recur:
                    pbuf_v[i, pl.ds(j * 16, 16)] = 2.0 * a - pbuf_v[i, pl.ds(j * 16, 16)]
                else:
                    pbuf_v[i, pl.ds(j * 16, 16)] = a
            return c
        lax.fori_loop(0, FBLK, fbody, 0)
        pltpu.sync_copy(pbuf_v, y_hbm.at[pl.ds(rb, FBLK)])


def _make_spmm(recur):
    mesh = plsc.VectorSubcoreMesh(core_axis_name="c", subcore_axis_name="s")
    return pl.kernel(
        functools.partial(_spmm_body, recur),
        out_type=jax.ShapeDtypeStruct((VPAD, F), jnp.float32),
        mesh=mesh,
        scratch_types=[
            pltpu.VMEM((16,), jnp.int32),
            pltpu.VMEM((CH_E,), jnp.int32),
            pltpu.VMEM((CH_E,), jnp.int32),
            pltpu.VMEM((CH_E + 16,), jnp.int32),
            pltpu.VMEM((CH_E + 16,), jnp.int32),
            pltpu.VMEM((CH_E + 16,), jnp.float32),
            pltpu.VMEM((CH_E + 16,), jnp.float32),
            pltpu.VMEM((CH_E, F), jnp.float32),
            pltpu.VMEM((CH_E, F), jnp.float32),
            pltpu.VMEM((RPW * F,), jnp.float32),
            pltpu.VMEM((FBLK, F), jnp.float32),
            pltpu.SemaphoreType.DMA,
            pltpu.SemaphoreType.DMA,
            pltpu.SemaphoreType.DMA,
            pltpu.SemaphoreType.DMA,
            pltpu.SemaphoreType.DMA,
            pltpu.SemaphoreType.DMA,
        ],
        name=f"sc_spmm_recur{int(recur)}",
    )


_spmm_first = _make_spmm(False)
_spmm_recur = _make_spmm(True)


NCHP = E // 128


def _prep1_body(row_hbm, col_hbm, ew_hbm, perm_hbm, z_hbm,
                rs_hbm, cs_hbm, eo_hbm, ewp_hbm, deg_hbm,
                pidx, rb, cb, eb, obuf, deg_sh, s0, s1, s2):
    cid = lax.axis_index("c")
    sid = lax.axis_index("s")
    wid = sid * 2 + cid

    @pl.when(sid == 0)
    def _():
        pltpu.sync_copy(z_hbm, deg_sh)
    plsc.subcore_barrier()

    nch = lax.shift_right_arithmetic(NCHP - wid + 31, 5)

    def body(k, c):
        st = pl.multiple_of((wid + k * 32) * 128, 128)
        pltpu.sync_copy(perm_hbm.at[pl.ds(st, 128)], pidx)
        pltpu.async_copy(row_hbm.at[pidx], rb, s0)
        pltpu.async_copy(col_hbm.at[pidx], cb, s1)
        pltpu.async_copy(ew_hbm.at[pidx], eb, s2)
        pltpu.make_async_copy(row_hbm.at[pidx], rb, s0).wait()
        pltpu.make_async_copy(col_hbm.at[pidx], cb, s1).wait()
        pltpu.make_async_copy(ew_hbm.at[pidx], eb, s2).wait()
        for g in range(8):
            r16 = rb[pl.ds(g * 16, 16)]
            obuf[pl.ds(g * 16, 16)] = lax.rem(r16, RPW) * F
        pltpu.sync_copy(eb, deg_sh.at[rb], add=True)
        pltpu.sync_copy(rb, rs_hbm.at[pl.ds(st, 128)])
        pltpu.sync_copy(cb, cs_hbm.at[pl.ds(st, 128)])
        pltpu.sync_copy(eb, ewp_hbm.at[pl.ds(st, 128)])
        pltpu.sync_copy(obuf, eo_hbm.at[pl.ds(st, 128)])
        return c
    lax.fori_loop(0, nch, body, 0)

    @pl.when(wid == NW - 1)
    def _():
        for g in range(8):
            obuf[pl.ds(g * 16, 16)] = jnp.zeros((16,), jnp.int32)
            pidx[pl.ds(g * 16, 16)] = jnp.full((16,), VPAD, jnp.int32)
        for pc in range((EPAD - E) // 128):
            pst = E + pc * 128
            pltpu.sync_copy(obuf, cs_hbm.at[pl.ds(pst, 128)])
            pltpu.sync_copy(obuf, eo_hbm.at[pl.ds(pst, 128)])
            pltpu.sync_copy(pidx, rs_hbm.at[pl.ds(pst, 128)])

    plsc.subcore_barrier()

    @pl.when(sid == 0)
    def _():
        pltpu.sync_copy(deg_sh, deg_hbm.at[cid])


_prep1 = pl.kernel(
    _prep1_body,
    out_type=[jax.ShapeDtypeStruct((EPAD,), jnp.int32),
              jax.ShapeDtypeStruct((EPAD,), jnp.int32),
              jax.ShapeDtypeStruct((EPAD,), jnp.int32),
              jax.ShapeDtypeStruct((EPAD,), jnp.float32),
              jax.ShapeDtypeStruct((2, VPAD), jnp.float32)],
    mesh=plsc.VectorSubcoreMesh(core_axis_name="c", subcore_axis_name="s"),
    scratch_types=[
        pltpu.VMEM((128,), jnp.int32),
        pltpu.VMEM((128,), jnp.int32),
        pltpu.VMEM((128,), jnp.int32),
        pltpu.VMEM((128,), jnp.float32),
        pltpu.VMEM((128,), jnp.int32),
        pltpu.VMEM_SHARED((VPAD,), jnp.float32),
        pltpu.SemaphoreType.DMA,
        pltpu.SemaphoreType.DMA,
        pltpu.SemaphoreType.DMA,
    ],
    name="sc_prep1",
)


def _prep2_body(rs_hbm, cs_hbm, ewp_hbm, dis_hbm, ws_hbm,
                rb, cb, eb, da, db, wbuf, s0, s1, s2, s3, s4):
    cid = lax.axis_index("c")
    sid = lax.axis_index("s")
    wid = sid * 2 + cid
    nch = lax.shift_right_arithmetic(NCHP - wid + 31, 5)

    def body(k, c):
        st = pl.multiple_of((wid + k * 32) * 128, 128)
        pltpu.async_copy(rs_hbm.at[pl.ds(st, 128)], rb, s0)
        pltpu.async_copy(cs_hbm.at[pl.ds(st, 128)], cb, s1)
        pltpu.async_copy(ewp_hbm.at[pl.ds(st, 128)], eb, s2)
        pltpu.make_async_copy(rs_hbm.at[pl.ds(st, 128)], rb, s0).wait()
        pltpu.make_async_copy(cs_hbm.at[pl.ds(st, 128)], cb, s1).wait()
        pltpu.async_copy(dis_hbm.at[rb], da, s3)
        pltpu.async_copy(dis_hbm.at[cb], db, s4)
        pltpu.make_async_copy(ewp_hbm.at[pl.ds(st, 128)], eb, s2).wait()
        pltpu.make_async_copy(dis_hbm.at[rb], da, s3).wait()
        pltpu.make_async_copy(dis_hbm.at[cb], db, s4).wait()
        for g in range(8):
            sl = pl.ds(g * 16, 16)
            wbuf[sl] = -(da[sl] * eb[sl] * db[sl])
        pltpu.sync_copy(wbuf, ws_hbm.at[pl.ds(st, 128)])
        return c
    lax.fori_loop(0, nch, body, 0)


_prep2 = pl.kernel(
    _prep2_body,
    out_type=jax.ShapeDtypeStruct((EPAD,), jnp.float32),
    mesh=plsc.VectorSubcoreMesh(core_axis_name="c", subcore_axis_name="s"),
    scratch_types=[
        pltpu.VMEM((128,), jnp.int32),
        pltpu.VMEM((128,), jnp.int32),
        pltpu.VMEM((128,), jnp.float32),
        pltpu.VMEM((128,), jnp.float32),
        pltpu.VMEM((128,), jnp.float32),
        pltpu.VMEM((128,), jnp.float32),
        pltpu.SemaphoreType.DMA,
        pltpu.SemaphoreType.DMA,
        pltpu.SemaphoreType.DMA,
        pltpu.SemaphoreType.DMA,
        pltpu.SemaphoreType.DMA,
    ],
    name="sc_prep2",
)


def _combine_body(has_res, refs):
    if has_res:
        (t0, t1, t2, t3, wbd, bias, res, out, s1, s2, mx) = refs
    else:
        (t0, t1, t2, t3, wbd, bias, out, s1, s2, mx) = refs
    o = bias[...]
    o = o + jnp.dot(t0[...], wbd[0], preferred_element_type=jnp.float32)
    o = o + jnp.dot(t1[...], wbd[1], preferred_element_type=jnp.float32)
    o = o + jnp.dot(t2[...], wbd[2], preferred_element_type=jnp.float32)
    o = o + jnp.dot(t3[...], wbd[3], preferred_element_type=jnp.float32)
    if has_res:
        o = o + res[...]
    o = jnp.maximum(o, 0.0)
    out[...] = o

    i = pl.program_id(0)
    rowid = lax.broadcasted_iota(jnp.int32, (RBLK, 1), 0) + i * RBLK
    valid = rowid < V
    om = jnp.where(valid, o, 0.0)
    ps1 = jnp.sum(om, axis=0, keepdims=True)
    ps2 = jnp.sum(om * om, axis=0, keepdims=True)
    pmx = jnp.max(jnp.where(valid, o, -jnp.inf), axis=0, keepdims=True)

    @pl.when(i == 0)
    def _():
        s1[...] = jnp.zeros_like(s1)
        s2[...] = jnp.zeros_like(s2)
        mx[...] = jnp.full_like(mx, -jnp.inf)

    s1[...] += ps1
    s2[...] += ps2
    mx[...] = jnp.maximum(mx[...], pmx)


def _make_combine(has_res):
    blk = pl.BlockSpec((RBLK, F), lambda i: (i, 0))
    small = pl.BlockSpec((1, F), lambda i: (0, 0))
    in_specs = [blk, blk, blk, blk,
                pl.BlockSpec((4, F, F), lambda i: (0, 0, 0)), small]
    if has_res:
        in_specs.append(blk)
    return pl.pallas_call(
        lambda *refs: _combine_body(has_res, refs),
        grid=(NBLK,),
        in_specs=in_specs,
        out_specs=[blk, small, small, small],
        out_shape=[jax.ShapeDtypeStruct((VPAD, F), jnp.float32),
                   jax.ShapeDtypeStruct((1, F), jnp.float32),
                   jax.ShapeDtypeStruct((1, F), jnp.float32),
                   jax.ShapeDtypeStruct((1, F), jnp.float32)],
        name=f"tc_combine_res{int(has_res)}",
    )


_combine_plain = _make_combine(False)
_combine_res = _make_combine(True)


def _bnapply_body(x, scale, shift, out):
    out[...] = x[...] * scale[...] + shift[...]


_bnapply = pl.pallas_call(
    _bnapply_body,
    grid=(NBLK,),
    in_specs=[pl.BlockSpec((RBLK, F), lambda i: (i, 0)),
              pl.BlockSpec((1, F), lambda i: (0, 0)),
              pl.BlockSpec((1, F), lambda i: (0, 0))],
    out_specs=pl.BlockSpec((RBLK, F), lambda i: (i, 0)),
    out_shape=jax.ShapeDtypeStruct((VPAD, F), jnp.float32),
    name="tc_bnapply",
)


def _head_body(mx, g, b, lw, lb, out):
    xs = [mx[:, pl.ds(i * C, C)] for i in range(B)]
    m = (xs[0] + xs[1] + xs[2] + xs[3]) * 0.25
    v = (xs[0] * xs[0] + xs[1] * xs[1] + xs[2] * xs[2] + xs[3] * xs[3]) * 0.25
    v = v - m * m
    inv = lax.rsqrt(v + 1e-5)
    rows = []
    for i in range(B):
        xh = (xs[i] - m) * inv * g[...] + b[...]
        logit = jnp.dot(xh, lw[...], preferred_element_type=jnp.float32) + lb[...]
        logit = jnp.maximum(logit, 0.0)
        z = logit - jnp.max(logit, axis=1, keepdims=True)
        rows.append(z - jnp.log(jnp.sum(jnp.exp(z), axis=1, keepdims=True)))
    out[...] = jnp.concatenate(rows, axis=0)


_head = pl.pallas_call(
    _head_body,
    out_shape=jax.ShapeDtypeStruct((B, 10), jnp.float32),
    name="tc_head",
)


def _cheb(xf, cs, eo, ew, offp, wbd, bias, res):
    t1 = _spmm_first(xf, xf, cs, eo, ew, offp)
    t2 = _spmm_recur(t1, xf, cs, eo, ew, offp)
    t3 = _spmm_recur(t2, t1, cs, eo, ew, offp)
    if res is None:
        return _combine_plain(xf, t1, t2, t3, wbd, bias)
    return _combine_res(xf, t1, t2, t3, wbd, bias, res)


def _bn_scale_shift(s1, s2, g, b):
    cnt = float(B * V)
    s1f = s1.reshape(B, C).sum(axis=0)
    s2f = s2.reshape(B, C).sum(axis=0)
    m = s1f / cnt
    var = s2f / cnt - m * m
    inv = lax.rsqrt(var + 1e-5)
    scale = inv * g
    shift = b - m * scale
    return (jnp.tile(scale, B).reshape(1, F),
            jnp.tile(shift, B).reshape(1, F))


def kernel(x, edge_index, edge_weight, in_conv_w, in_conv_b,
           b0_in_bn_g, b0_in_bn_b, b0_in_conv_w, b0_in_conv_b,
           b0_h_bn_g, b0_h_bn_b, b0_h_conv_w, b0_h_conv_b,
           b1_in_bn_g, b1_in_bn_b, b1_in_conv_w, b1_in_conv_b,
           b1_h_bn_g, b1_h_bn_b, b1_h_conv_w, b1_h_conv_b,
           out_bn_g, out_bn_b, lin_w, lin_b):
    row = edge_index[0].astype(jnp.int32)
    col = edge_index[1].astype(jnp.int32)

    perm = jnp.argsort(row).astype(jnp.int32)
    zvec = jnp.zeros((VPAD,), jnp.float32)
    rs, cs, eo, ewp, deg2 = _prep1(row, col, edge_weight, perm, zvec)
    dis = lax.rsqrt(jnp.maximum(deg2[0] + deg2[1], 1e-12))
    ew = _prep2(rs, cs, ewp, dis)
    bounds = jnp.arange(0, VPAD + 1, RPW, dtype=jnp.int32)
    off = jnp.searchsorted(rs, bounds).astype(jnp.int32)
    offp = jnp.zeros((NW, 16), jnp.int32)
    offp = offp.at[:, 0].set(off[:-1]).at[:, 1].set(off[1:])

    xf = jnp.transpose(x, (2, 0, 1)).reshape(V, F)
    xf = jnp.concatenate([xf, jnp.zeros((VPAD - V, F), jnp.float32)])

    eye = jnp.eye(B, dtype=jnp.float32)
    def bd(w):
        return jnp.einsum('ab,kcd->kacbd', eye, w).reshape(w.shape[0], F, F)

    def bias128(bv):
        return jnp.tile(bv, B).reshape(1, F)

    out0, s1, s2, _ = _cheb(xf, cs, eo, ew, offp, bd(in_conv_w),
                            bias128(in_conv_b), None)
    sc, sh = _bn_scale_shift(s1, s2, b0_in_bn_g, b0_in_bn_b)
    xn = _bnapply(out0, sc, sh)
    c1, s1, s2, _ = _cheb(xn, cs, eo, ew, offp, bd(b0_in_conv_w),
                          bias128(b0_in_conv_b), None)
    sc, sh = _bn_scale_shift(s1, s2, b0_h_bn_g, b0_h_bn_b)
    c1n = _bnapply(c1, sc, sh)
    out1, s1, s2, _ = _cheb(c1n, cs, eo, ew, offp, bd(b0_h_conv_w),
                            bias128(b0_h_conv_b), xn)
    sc, sh = _bn_scale_shift(s1, s2, b1_in_bn_g, b1_in_bn_b)
    xn = _bnapply(out1, sc, sh)
    c1, s1, s2, _ = _cheb(xn, cs, eo, ew, offp, bd(b1_in_conv_w),
                          bias128(b1_in_conv_b), None)
    sc, sh = _bn_scale_shift(s1, s2, b1_h_bn_g, b1_h_bn_b)
    c1n = _bnapply(c1, sc, sh)
    _, _, _, mx = _cheb(c1n, cs, eo, ew, offp, bd(b1_h_conv_w),
                        bias128(b1_h_conv_b), xn)
    return _head(mx, out_bn_g.reshape(1, C), out_bn_b.reshape(1, C),
                 lin_w, lin_b.reshape(1, 10))
